# baseline (speedup 1.0000x reference)
import functools

import jax
import jax.numpy as jnp
from jax import lax
from jax.experimental import pallas as pl
from jax.experimental.pallas import tpu as pltpu
from jax.experimental.pallas import tpu_sc as plsc

B = 4096
N = 4096
C = 6
NB = 12
TIN = C + N
S = 128
SU = 80
BB = 512
NTILES = 16
K = 5
HB = B // 2
TB = HB // NTILES


def _addr_tc_kernel(prev_ref, char_ref, cst_ref, cof_ref, out_ref,
                    wlo_ref, whi_ref, wc_ref):
    @pl.when(pl.program_id(0) == 0)
    def _build_weights():
        niota = lax.broadcasted_iota(jnp.int32, (N, S), 0)
        m = (niota == cof_ref[...]).astype(jnp.float32)
        cselt = jnp.dot(cst_ref[...], m, preferred_element_type=jnp.float32,
                        precision=lax.Precision.HIGHEST).astype(jnp.int32)
        uio = lax.broadcasted_iota(jnp.int32, (8, S), 0)
        wlo = jnp.zeros((N, S), jnp.float32)
        whi = jnp.zeros((N, S), jnp.float32)
        wc = jnp.zeros((8, S), jnp.float32)
        for k in range(NB):
            row = cselt[k:k + 1, :]
            sm = (niota == row - C).astype(jnp.float32)
            if k < 8:
                wlo = wlo + float(2 ** k) * sm
            else:
                whi = whi + float(2 ** (k - 8)) * sm
            cm = jnp.logical_and(uio == row, uio < C)
            wc = wc + float(2 ** k) * cm.astype(jnp.float32)
        wlo_ref[...] = wlo.astype(jnp.bfloat16)
        whi_ref[...] = whi.astype(jnp.bfloat16)
        wc_ref[...] = wc

    pv = prev_ref[...].astype(jnp.bfloat16)
    dn = (((0,), (1,)), ((), ()))
    alo = lax.dot_general(wlo_ref[...], pv, dn, preferred_element_type=jnp.float32)
    ahi = lax.dot_general(whi_ref[...], pv, dn, preferred_element_type=jnp.float32)
    ac = lax.dot_general(wc_ref[...], char_ref[...], dn, preferred_element_type=jnp.float32,
                         precision=lax.Precision.HIGHEST)
    out_ref[...] = (alo + 256.0 * ahi + ac).astype(jnp.int32)


def _rowgather_tc_kernel(cof_ref, smem_ref, out_ref):
    out_ref[...] = smem_ref[...]


def _sc_kernel(addrt_hbm, tabs_hbm, tgt_hbm, rows8_hbm, out_hbm,
               tabs_v, addrb_v, contrib_v, lcnt_v, ltsum_v,
               ao_v, tg_v, rows_v, acc_sh, cnt_sh, tsum_sh, sem):
    core = lax.axis_index("c")
    tid = lax.axis_index("s")
    b0 = core * HB
    s0 = tid * K

    zf16 = jnp.zeros((16,), jnp.float32)

    def _zero(ref, ncols):
        for r in range(8):
            def body(i, _):
                ref[r, pl.ds(i * 16, 16)] = zf16
                return 0
            lax.fori_loop(0, ncols // 16, body, 0)

    _zero(contrib_v, HB)
    _zero(lcnt_v, N)
    _zero(ltsum_v, N)

    @pl.when(tid == 0)
    def _init_shared():
        pltpu.sync_copy(contrib_v, acc_sh)
        pltpu.sync_copy(lcnt_v, cnt_sh)
        pltpu.sync_copy(lcnt_v, tsum_sh)

    pltpu.sync_copy(rows8_hbm, rows_v)
    for k in range(K):
        pltpu.sync_copy(tabs_hbm.at[pl.ds(s0 + k, 1)], tabs_v.at[pl.ds(k, 1)])
        pltpu.sync_copy(addrt_hbm.at[pl.ds(s0 + k, 1), pl.ds(b0, HB)],
                        addrb_v.at[pl.ds(k, 1)])

    plsc.subcore_barrier()

    iota16 = lax.iota(jnp.int32, 16)
    for k in range(K):
        sg = s0 + k
        j = lax.rem(sg, NB)
        c = lax.div(sg, NB)
        wf = lax.shift_left(jnp.int32(1), j).astype(jnp.float32)
        kvec = jnp.full((16,), k, jnp.int32)
        cvec = jnp.broadcast_to(c, (16,))

        def body(i, _):
            off = i * 16
            idx = addrb_v[k, pl.ds(off, 16)]
            v = plsc.load_gather(tabs_v, [kvec, idx])
            contrib = jnp.where(v > 0.5, wf, 0.0)
            plsc.addupdate_scatter(contrib_v, [cvec, off + iota16], contrib)
            return 0

        lax.fori_loop(0, HB // 16, body, 0)

    pltpu.sync_copy(contrib_v, acc_sh.at[rows_v], add=True)
    plsc.subcore_barrier()

    tb0 = tid * TB
    ones16 = jnp.ones((16,), jnp.float32)
    for c in range(C):
        pltpu.sync_copy(acc_sh.at[pl.ds(c, 1), pl.ds(tb0, TB)], ao_v)
        pltpu.sync_copy(tgt_hbm.at[pl.ds(c, 1), pl.ds(b0 + tb0, TB)], tg_v)
        ccol = jnp.full((16,), c, jnp.int32)
        for i in range(TB // 16):
            idx = ao_v[0, pl.ds(i * 16, 16)].astype(jnp.int32)
            plsc.addupdate_scatter(lcnt_v, [ccol, idx], ones16)
            plsc.addupdate_scatter(ltsum_v, [ccol, idx], tg_v[0, pl.ds(i * 16, 16)])

    pltpu.sync_copy(lcnt_v, cnt_sh.at[rows_v], add=True)
    pltpu.sync_copy(ltsum_v, tsum_sh.at[rows_v], add=True)
    plsc.subcore_barrier()

    @pl.when(tid == 0)
    def _writeout():
        pltpu.sync_copy(cnt_sh, out_hbm.at[core * 2])
        pltpu.sync_copy(tsum_sh, out_hbm.at[core * 2 + 1])


def _combine_tc_kernel(om_ref, part_ref, out_ref):
    cnt = part_ref[0, :C, :] + part_ref[2, :C, :]
    ts = part_ref[1, :C, :] + part_ref[3, :C, :]
    om = om_ref[...]
    out_ref[...] = om * (1.0 - cnt) + ts


def kernel(target_bits, state_mem, out_mem, char_bits, prev_state, conn_state, conn_out):
    f32 = jnp.float32

    cof = conn_out.reshape(-1)
    cof_row = jnp.pad(cof, (0, S - cof.shape[0]))[None, :]
    cof80 = jnp.pad(cof, (0, SU - cof.shape[0]))
    cst = jnp.pad(conn_state.T.astype(f32), ((0, 4), (0, 0)))
    char_pad = jnp.pad(char_bits, ((0, 0), (0, 2))).astype(f32)
    tgt_t = jnp.pad(target_bits.T, ((0, 2), (0, 0)))
    rows8 = jnp.arange(8, dtype=jnp.int32)

    addrt = pl.pallas_call(
        _addr_tc_kernel,
        grid=(B // BB,),
        in_specs=[
            pl.BlockSpec((BB, N), lambda i: (i, 0)),
            pl.BlockSpec((BB, 8), lambda i: (i, 0)),
            pl.BlockSpec((16, N), lambda i: (0, 0)),
            pl.BlockSpec((1, S), lambda i: (0, 0)),
        ],
        out_specs=pl.BlockSpec((S, BB), lambda i: (0, i)),
        out_shape=jax.ShapeDtypeStruct((S, B), jnp.int32),
        scratch_shapes=[
            pltpu.VMEM((N, S), jnp.bfloat16),
            pltpu.VMEM((N, S), jnp.bfloat16),
            pltpu.VMEM((8, S), f32),
        ],
    )(prev_state, char_pad, cst, cof_row)

    tabs80 = pl.pallas_call(
        _rowgather_tc_kernel,
        grid_spec=pltpu.PrefetchScalarGridSpec(
            num_scalar_prefetch=1,
            grid=(SU,),
            in_specs=[
                pl.BlockSpec((1, 1, N), lambda i, cof_ref: (cof_ref[i], 0, 0)),
            ],
            out_specs=pl.BlockSpec((1, 1, N), lambda i, cof_ref: (i, 0, 0)),
        ),
        out_shape=jax.ShapeDtypeStruct((SU, 1, N), f32),
    )(cof80, state_mem.reshape(N, 1, N)).reshape(SU, N)

    mesh = plsc.VectorSubcoreMesh(core_axis_name="c", subcore_axis_name="s")
    sc_call = pl.kernel(
        _sc_kernel,
        out_type=jax.ShapeDtypeStruct((4, 8, N), f32),
        mesh=mesh,
        compiler_params=pltpu.CompilerParams(use_tc_tiling_on_sc=False,
                                             needs_layout_passes=False),
        scratch_types=[
            pltpu.VMEM((K, N), f32),
            pltpu.VMEM((K, HB), jnp.int32),
            pltpu.VMEM((8, HB), f32),
            pltpu.VMEM((8, N), f32),
            pltpu.VMEM((8, N), f32),
            pltpu.VMEM((1, TB), f32),
            pltpu.VMEM((1, TB), f32),
            pltpu.VMEM((8,), jnp.int32),
            pltpu.VMEM_SHARED((8, HB), f32),
            pltpu.VMEM_SHARED((8, N), f32),
            pltpu.VMEM_SHARED((8, N), f32),
            pltpu.SemaphoreType.DMA,
        ],
    )
    partials = sc_call(addrt, tabs80, tgt_t, rows8)

    new_out = pl.pallas_call(
        _combine_tc_kernel,
        out_shape=jax.ShapeDtypeStruct((C, N), f32),
    )(out_mem, partials)
    return new_out


# speedup vs baseline: 1.8786x; 1.8786x over previous
import functools

import jax
import jax.numpy as jnp
from jax import lax
from jax.experimental import pallas as pl
from jax.experimental.pallas import tpu as pltpu
from jax.experimental.pallas import tpu_sc as plsc

B = 4096
N = 4096
C = 6
NB = 12
TIN = C + N
S = 128
SU = 80
BB = 512
NTILES = 16
K = 5
HB = B // 2
TB = HB // NTILES


RPS = S // (B // BB)


def _addr_tc_kernel(prev_ref, char_ref, cst_ref, cof_ref, cof80_ref, smem_ref,
                    out_ref, tabs_ref, wlo_ref, whi_ref, wc_ref, sem):
    pid = pl.program_id(0)
    copies = [
        pltpu.make_async_copy(
            smem_ref.at[pl.ds(cof80_ref[pid * RPS + k], 1), :],
            tabs_ref.at[pl.ds(k, 1), :], sem)
        for k in range(RPS)
    ]
    for cp in copies:
        cp.start()

    @pl.when(pl.program_id(0) == 0)
    def _build_weights():
        niota = lax.broadcasted_iota(jnp.int32, (N, S), 0)
        m = (niota == cof_ref[...]).astype(jnp.float32)
        cselt = jnp.dot(cst_ref[...], m, preferred_element_type=jnp.float32,
                        precision=lax.Precision.HIGHEST).astype(jnp.int32)
        uio = lax.broadcasted_iota(jnp.int32, (8, S), 0)
        wlo = jnp.zeros((N, S), jnp.float32)
        whi = jnp.zeros((N, S), jnp.float32)
        wc = jnp.zeros((8, S), jnp.float32)
        for k in range(NB):
            row = cselt[k:k + 1, :]
            sm = (niota == row - C).astype(jnp.float32)
            if k < 8:
                wlo = wlo + float(2 ** k) * sm
            else:
                whi = whi + float(2 ** (k - 8)) * sm
            cm = jnp.logical_and(uio == row, uio < C)
            wc = wc + float(2 ** k) * cm.astype(jnp.float32)
        wlo_ref[...] = wlo.astype(jnp.bfloat16)
        whi_ref[...] = whi.astype(jnp.bfloat16)
        wc_ref[...] = wc

    pv = prev_ref[...].astype(jnp.bfloat16)
    dn = (((0,), (1,)), ((), ()))
    alo = lax.dot_general(wlo_ref[...], pv, dn, preferred_element_type=jnp.float32)
    ahi = lax.dot_general(whi_ref[...], pv, dn, preferred_element_type=jnp.float32)
    ac = lax.dot_general(wc_ref[...], char_ref[...], dn, preferred_element_type=jnp.float32,
                         precision=lax.Precision.HIGHEST)
    out_ref[...] = (alo + 256.0 * ahi + ac).astype(jnp.int32)
    for cp in copies:
        cp.wait()


def _sc_kernel(addrt_hbm, tabs_hbm, tgt_hbm, rows8_hbm, out_hbm,
               tabs_v, addrb_v, contrib_v, lcnt_v, ltsum_v,
               ao_v, tg_v, rows_v, acc_sh, cnt_sh, tsum_sh, sem):
    core = lax.axis_index("c")
    tid = lax.axis_index("s")
    b0 = core * HB
    s0 = tid * K

    zf16 = jnp.zeros((16,), jnp.float32)

    def _zero(ref, ncols):
        for r in range(8):
            def body(i, _):
                ref[r, pl.ds(i * 16, 16)] = zf16
                return 0
            lax.fori_loop(0, ncols // 16, body, 0)

    _zero(contrib_v, HB)
    _zero(lcnt_v, N)
    _zero(ltsum_v, N)

    @pl.when(tid == 0)
    def _init_shared():
        pltpu.sync_copy(contrib_v, acc_sh)
        pltpu.sync_copy(lcnt_v, cnt_sh)
        pltpu.sync_copy(lcnt_v, tsum_sh)

    pltpu.sync_copy(rows8_hbm, rows_v)
    for k in range(K):
        pltpu.sync_copy(tabs_hbm.at[pl.ds(s0 + k, 1)], tabs_v.at[pl.ds(k, 1)])
        pltpu.sync_copy(addrt_hbm.at[pl.ds(s0 + k, 1), pl.ds(b0, HB)],
                        addrb_v.at[pl.ds(k, 1)])

    plsc.subcore_barrier()

    iota16 = lax.iota(jnp.int32, 16)
    for k in range(K):
        sg = s0 + k
        j = lax.rem(sg, NB)
        c = lax.div(sg, NB)
        wf = lax.shift_left(jnp.int32(1), j).astype(jnp.float32)
        kvec = jnp.full((16,), k, jnp.int32)
        cvec = jnp.broadcast_to(c, (16,))

        def body(i, _):
            off = i * 16
            idx = addrb_v[k, pl.ds(off, 16)]
            v = plsc.load_gather(tabs_v, [kvec, idx])
            contrib = jnp.where(v > 0.5, wf, 0.0)
            plsc.addupdate_scatter(contrib_v, [cvec, off + iota16], contrib)
            return 0

        lax.fori_loop(0, HB // 16, body, 0)

    pltpu.sync_copy(contrib_v, acc_sh.at[rows_v], add=True)
    plsc.subcore_barrier()

    tb0 = tid * TB
    ones16 = jnp.ones((16,), jnp.float32)
    for c in range(C):
        pltpu.sync_copy(acc_sh.at[pl.ds(c, 1), pl.ds(tb0, TB)], ao_v)
        pltpu.sync_copy(tgt_hbm.at[pl.ds(c, 1), pl.ds(b0 + tb0, TB)], tg_v)
        ccol = jnp.full((16,), c, jnp.int32)
        for i in range(TB // 16):
            idx = ao_v[0, pl.ds(i * 16, 16)].astype(jnp.int32)
            plsc.addupdate_scatter(lcnt_v, [ccol, idx], ones16)
            plsc.addupdate_scatter(ltsum_v, [ccol, idx], tg_v[0, pl.ds(i * 16, 16)])

    pltpu.sync_copy(lcnt_v, cnt_sh.at[rows_v], add=True)
    pltpu.sync_copy(ltsum_v, tsum_sh.at[rows_v], add=True)
    plsc.subcore_barrier()

    @pl.when(tid == 0)
    def _writeout():
        pltpu.sync_copy(cnt_sh, out_hbm.at[core * 2])
        pltpu.sync_copy(tsum_sh, out_hbm.at[core * 2 + 1])


def _combine_tc_kernel(om_ref, part_ref, out_ref):
    cnt = part_ref[0, :C, :] + part_ref[2, :C, :]
    ts = part_ref[1, :C, :] + part_ref[3, :C, :]
    om = om_ref[...]
    out_ref[...] = om * (1.0 - cnt) + ts


def kernel(target_bits, state_mem, out_mem, char_bits, prev_state, conn_state, conn_out):
    f32 = jnp.float32

    cof = conn_out.reshape(-1)
    cof128 = jnp.pad(cof, (0, S - cof.shape[0]))
    cof_row = cof128[None, :]
    cst = jnp.pad(conn_state.T.astype(f32), ((0, 4), (0, 0)))
    char_pad = jnp.pad(char_bits, ((0, 0), (0, 2))).astype(f32)
    tgt_t = jnp.pad(target_bits.T, ((0, 2), (0, 0)))
    rows8 = jnp.arange(8, dtype=jnp.int32)

    addrt, tabs80 = pl.pallas_call(
        _addr_tc_kernel,
        grid=(B // BB,),
        in_specs=[
            pl.BlockSpec((BB, N), lambda i: (i, 0)),
            pl.BlockSpec((BB, 8), lambda i: (i, 0)),
            pl.BlockSpec((16, N), lambda i: (0, 0)),
            pl.BlockSpec((1, S), lambda i: (0, 0)),
            pl.BlockSpec(memory_space=pltpu.MemorySpace.SMEM),
            pl.BlockSpec(memory_space=pltpu.MemorySpace.HBM),
        ],
        out_specs=[
            pl.BlockSpec((S, BB), lambda i: (0, i)),
            pl.BlockSpec((RPS, N), lambda i: (i, 0)),
        ],
        out_shape=[
            jax.ShapeDtypeStruct((S, B), jnp.int32),
            jax.ShapeDtypeStruct((S, N), f32),
        ],
        scratch_shapes=[
            pltpu.VMEM((N, S), jnp.bfloat16),
            pltpu.VMEM((N, S), jnp.bfloat16),
            pltpu.VMEM((8, S), f32),
            pltpu.SemaphoreType.DMA,
        ],
    )(prev_state, char_pad, cst, cof_row, cof128, state_mem)

    mesh = plsc.VectorSubcoreMesh(core_axis_name="c", subcore_axis_name="s")
    sc_call = pl.kernel(
        _sc_kernel,
        out_type=jax.ShapeDtypeStruct((4, 8, N), f32),
        mesh=mesh,
        compiler_params=pltpu.CompilerParams(use_tc_tiling_on_sc=False,
                                             needs_layout_passes=False),
        scratch_types=[
            pltpu.VMEM((K, N), f32),
            pltpu.VMEM((K, HB), jnp.int32),
            pltpu.VMEM((8, HB), f32),
            pltpu.VMEM((8, N), f32),
            pltpu.VMEM((8, N), f32),
            pltpu.VMEM((1, TB), f32),
            pltpu.VMEM((1, TB), f32),
            pltpu.VMEM((8,), jnp.int32),
            pltpu.VMEM_SHARED((8, HB), f32),
            pltpu.VMEM_SHARED((8, N), f32),
            pltpu.VMEM_SHARED((8, N), f32),
            pltpu.SemaphoreType.DMA,
        ],
    )
    partials = sc_call(addrt, tabs80, tgt_t, rows8)

    new_out = pl.pallas_call(
        _combine_tc_kernel,
        out_shape=jax.ShapeDtypeStruct((C, N), f32),
    )(out_mem, partials)
    return new_out


# speedup vs baseline: 1.9144x; 1.0191x over previous
import functools

import jax
import jax.numpy as jnp
from jax import lax
from jax.experimental import pallas as pl
from jax.experimental.pallas import tpu as pltpu
from jax.experimental.pallas import tpu_sc as plsc

B = 4096
N = 4096
C = 6
NB = 12
TIN = C + N
S = 128
SU = 80
BB = 512
NTILES = 16
K = 5
HB = B // 2
TB = HB // NTILES


RPS = S // (B // BB)


def _addr_tc_kernel(prev_ref, char_ref, cst_ref, cof_ref, cof80_ref, smem_ref,
                    out_ref, tabs_ref, wlo_ref, whi_ref, wc_ref, sem):
    pid = pl.program_id(0)
    copies = [
        pltpu.make_async_copy(
            smem_ref.at[pl.ds(cof80_ref[pid * RPS + k], 1), :],
            tabs_ref.at[pl.ds(k, 1), :], sem)
        for k in range(RPS)
    ]
    for cp in copies:
        cp.start()

    @pl.when(pl.program_id(0) == 0)
    def _build_weights():
        niota = lax.broadcasted_iota(jnp.int32, (N, S), 0)
        m = (niota == cof_ref[...]).astype(jnp.float32)
        cselt = jnp.dot(cst_ref[...], m, preferred_element_type=jnp.float32,
                        precision=lax.Precision.HIGHEST).astype(jnp.int32)
        uio = lax.broadcasted_iota(jnp.int32, (8, S), 0)
        wlo = jnp.zeros((N, S), jnp.float32)
        whi = jnp.zeros((N, S), jnp.float32)
        wc = jnp.zeros((8, S), jnp.float32)
        for k in range(NB):
            row = cselt[k:k + 1, :]
            sm = (niota == row - C).astype(jnp.float32)
            if k < 8:
                wlo = wlo + float(2 ** k) * sm
            else:
                whi = whi + float(2 ** (k - 8)) * sm
            cm = jnp.logical_and(uio == row, uio < C)
            wc = wc + float(2 ** k) * cm.astype(jnp.float32)
        wlo_ref[...] = wlo.astype(jnp.bfloat16)
        whi_ref[...] = whi.astype(jnp.bfloat16)
        wc_ref[...] = wc

    pv = prev_ref[...].astype(jnp.bfloat16)
    dn = (((0,), (1,)), ((), ()))
    alo = lax.dot_general(wlo_ref[...], pv, dn, preferred_element_type=jnp.float32)
    ahi = lax.dot_general(whi_ref[...], pv, dn, preferred_element_type=jnp.float32)
    ac = lax.dot_general(wc_ref[...], char_ref[...], dn, preferred_element_type=jnp.float32,
                         precision=lax.Precision.HIGHEST)
    out_ref[...] = (alo + 256.0 * ahi + ac).astype(jnp.int32)
    for cp in copies:
        cp.wait()


def _sc_kernel(addrt_hbm, tabs_hbm, tgt_hbm, rows8_hbm, zeros_hbm, out_hbm,
               tabs_v, addrb_v, contrib_v, lcnt_v, ltsum_v,
               ao_v, tg_v, rows_v, acc_sh, cnt_sh, tsum_sh, sem):
    core = lax.axis_index("c")
    tid = lax.axis_index("s")
    b0 = core * HB
    s0 = tid * K

    pltpu.sync_copy(zeros_hbm, lcnt_v)
    pltpu.sync_copy(zeros_hbm, ltsum_v)
    for r in range(8):
        pltpu.sync_copy(zeros_hbm.at[pl.ds(0, 1), pl.ds(0, HB)],
                        contrib_v.at[pl.ds(r, 1)])

    @pl.when(tid == 0)
    def _init_shared():
        pltpu.sync_copy(contrib_v, acc_sh)
        pltpu.sync_copy(lcnt_v, cnt_sh)
        pltpu.sync_copy(lcnt_v, tsum_sh)

    pltpu.sync_copy(rows8_hbm, rows_v)
    for k in range(K):
        pltpu.sync_copy(tabs_hbm.at[pl.ds(s0 + k, 1)], tabs_v.at[pl.ds(k, 1)])
        pltpu.sync_copy(addrt_hbm.at[pl.ds(s0 + k, 1), pl.ds(b0, HB)],
                        addrb_v.at[pl.ds(k, 1)])
    plsc.subcore_barrier()

    iota16 = lax.iota(jnp.int32, 16)
    UNR = 8
    for k in range(K):
        sg = s0 + k
        j = lax.rem(sg, NB)
        c = lax.div(sg, NB)
        wf = lax.shift_left(jnp.int32(1), j).astype(jnp.float32)
        kvec = jnp.full((16,), k, jnp.int32)
        cvec = jnp.broadcast_to(c, (16,))

        def body(i, _):
            for u in range(UNR):
                off = i * (16 * UNR) + u * 16
                idx = addrb_v[k, pl.ds(off, 16)]
                v = plsc.load_gather(tabs_v, [kvec, idx])
                contrib = jnp.where(v > 0.5, wf, 0.0)
                plsc.addupdate_scatter(contrib_v, [cvec, off + iota16], contrib)
            return 0

        lax.fori_loop(0, HB // (16 * UNR), body, 0)

    pltpu.sync_copy(contrib_v, acc_sh.at[rows_v], add=True)
    plsc.subcore_barrier()

    tb0 = tid * TB
    ones16 = jnp.ones((16,), jnp.float32)
    for c in range(C):
        pltpu.sync_copy(acc_sh.at[pl.ds(c, 1), pl.ds(tb0, TB)], ao_v.at[pl.ds(c, 1)])
        pltpu.sync_copy(tgt_hbm.at[pl.ds(c, 1), pl.ds(b0 + tb0, TB)], tg_v.at[pl.ds(c, 1)])
    for c in range(C):
        ccol = jnp.full((16,), c, jnp.int32)
        for i in range(TB // 16):
            idx = ao_v[c, pl.ds(i * 16, 16)].astype(jnp.int32)
            plsc.addupdate_scatter(lcnt_v, [ccol, idx], ones16)
            plsc.addupdate_scatter(ltsum_v, [ccol, idx], tg_v[c, pl.ds(i * 16, 16)])

    pltpu.sync_copy(lcnt_v, cnt_sh.at[rows_v], add=True)
    pltpu.sync_copy(ltsum_v, tsum_sh.at[rows_v], add=True)
    plsc.subcore_barrier()

    @pl.when(tid == 0)
    def _writeout():
        pltpu.sync_copy(cnt_sh, out_hbm.at[core * 2])
        pltpu.sync_copy(tsum_sh, out_hbm.at[core * 2 + 1])


def _combine_tc_kernel(om_ref, part_ref, out_ref):
    cnt = part_ref[0, :C, :] + part_ref[2, :C, :]
    ts = part_ref[1, :C, :] + part_ref[3, :C, :]
    om = om_ref[...]
    out_ref[...] = om * (1.0 - cnt) + ts


def kernel(target_bits, state_mem, out_mem, char_bits, prev_state, conn_state, conn_out):
    f32 = jnp.float32

    cof = conn_out.reshape(-1)
    cof128 = jnp.pad(cof, (0, S - cof.shape[0]))
    cof_row = cof128[None, :]
    cst = jnp.pad(conn_state.T.astype(f32), ((0, 4), (0, 0)))
    char_pad = jnp.pad(char_bits, ((0, 0), (0, 2))).astype(f32)
    tgt_t = jnp.pad(target_bits.T, ((0, 2), (0, 0)))
    rows8 = jnp.arange(8, dtype=jnp.int32)

    addrt, tabs80 = pl.pallas_call(
        _addr_tc_kernel,
        grid=(B // BB,),
        in_specs=[
            pl.BlockSpec((BB, N), lambda i: (i, 0)),
            pl.BlockSpec((BB, 8), lambda i: (i, 0)),
            pl.BlockSpec((16, N), lambda i: (0, 0)),
            pl.BlockSpec((1, S), lambda i: (0, 0)),
            pl.BlockSpec(memory_space=pltpu.MemorySpace.SMEM),
            pl.BlockSpec(memory_space=pltpu.MemorySpace.HBM),
        ],
        out_specs=[
            pl.BlockSpec((S, BB), lambda i: (0, i)),
            pl.BlockSpec((RPS, N), lambda i: (i, 0)),
        ],
        out_shape=[
            jax.ShapeDtypeStruct((S, B), jnp.int32),
            jax.ShapeDtypeStruct((S, N), f32),
        ],
        scratch_shapes=[
            pltpu.VMEM((N, S), jnp.bfloat16),
            pltpu.VMEM((N, S), jnp.bfloat16),
            pltpu.VMEM((8, S), f32),
            pltpu.SemaphoreType.DMA,
        ],
    )(prev_state, char_pad, cst, cof_row, cof128, state_mem)

    mesh = plsc.VectorSubcoreMesh(core_axis_name="c", subcore_axis_name="s")
    sc_call = pl.kernel(
        _sc_kernel,
        out_type=jax.ShapeDtypeStruct((4, 8, N), f32),
        mesh=mesh,
        compiler_params=pltpu.CompilerParams(use_tc_tiling_on_sc=False,
                                             needs_layout_passes=False),
        scratch_types=[
            pltpu.VMEM((K, N), f32),
            pltpu.VMEM((K, HB), jnp.int32),
            pltpu.VMEM((8, HB), f32),
            pltpu.VMEM((8, N), f32),
            pltpu.VMEM((8, N), f32),
            pltpu.VMEM((8, TB), f32),
            pltpu.VMEM((8, TB), f32),
            pltpu.VMEM((8,), jnp.int32),
            pltpu.VMEM_SHARED((8, HB), f32),
            pltpu.VMEM_SHARED((8, N), f32),
            pltpu.VMEM_SHARED((8, N), f32),
            pltpu.SemaphoreType.DMA,
        ],
    )
    zeros8n = jnp.zeros((8, N), f32)
    partials = sc_call(addrt, tabs80, tgt_t, rows8, zeros8n)

    new_out = pl.pallas_call(
        _combine_tc_kernel,
        out_shape=jax.ShapeDtypeStruct((C, N), f32),
    )(out_mem, partials)
    return new_out


# speedup vs baseline: 1.9842x; 1.0365x over previous
import functools

import jax
import jax.numpy as jnp
from jax import lax
from jax.experimental import pallas as pl
from jax.experimental.pallas import tpu as pltpu
from jax.experimental.pallas import tpu_sc as plsc

B = 4096
N = 4096
C = 6
NB = 12
TIN = C + N
S = 128
SU = 80
BB = 512
NTILES = 16
K = 5
HB = B // 2
TB = HB // NTILES


RPS = S // (B // BB)


def _addr_tc_kernel(prev_ref, char_ref, cst_ref, cof_ref, cof80_ref, smem_ref,
                    out_ref, tabs_ref, wlo_ref, whi_ref, wc_ref, sem):
    pid = pl.program_id(0)
    copies = [
        pltpu.make_async_copy(
            smem_ref.at[pl.ds(cof80_ref[pid * RPS + k], 1), :],
            tabs_ref.at[pl.ds(k, 1), :], sem)
        for k in range(RPS)
    ]
    for cp in copies:
        cp.start()

    @pl.when(pl.program_id(0) == 0)
    def _build_weights():
        niota = lax.broadcasted_iota(jnp.int32, (N, S), 0)
        m = (niota == cof_ref[...]).astype(jnp.float32)
        cselt = jnp.dot(cst_ref[...], m, preferred_element_type=jnp.float32,
                        precision=lax.Precision.HIGHEST).astype(jnp.int32)
        uio = lax.broadcasted_iota(jnp.int32, (8, S), 0)
        wlo = jnp.zeros((N, S), jnp.float32)
        whi = jnp.zeros((N, S), jnp.float32)
        wc = jnp.zeros((8, S), jnp.float32)
        for k in range(NB):
            row = cselt[k:k + 1, :]
            sm = (niota == row - C).astype(jnp.float32)
            if k < 8:
                wlo = wlo + float(2 ** k) * sm
            else:
                whi = whi + float(2 ** (k - 8)) * sm
            cm = jnp.logical_and(uio == row, uio < C)
            wc = wc + float(2 ** k) * cm.astype(jnp.float32)
        wlo_ref[...] = wlo.astype(jnp.bfloat16)
        whi_ref[...] = whi.astype(jnp.bfloat16)
        wc_ref[...] = wc

    pv = prev_ref[...].astype(jnp.bfloat16)
    dn = (((0,), (1,)), ((), ()))
    alo = lax.dot_general(wlo_ref[...], pv, dn, preferred_element_type=jnp.float32)
    ahi = lax.dot_general(whi_ref[...], pv, dn, preferred_element_type=jnp.float32)
    ac = lax.dot_general(wc_ref[...], char_ref[...], dn, preferred_element_type=jnp.float32,
                         precision=lax.Precision.HIGHEST)
    out_ref[...] = (alo + 256.0 * ahi + ac).astype(jnp.int32)
    for cp in copies:
        cp.wait()


AW = N // NTILES


def _sc_kernel(addrt_hbm, tabs_hbm, tgt_hbm, rowsel_hbm, zeros_hbm, out_hbm,
               tabs_v, addrb_v, contrib_v, lcnt_v, ltsum_v,
               ao_v, tg_v, rowsel_v, acc_sh, cnt_sh, tsum_sh, sem):
    core = lax.axis_index("c")
    tid = lax.axis_index("s")
    b0 = core * HB
    s0 = tid * K
    c_lo = lax.div(s0, NB)

    zf16 = jnp.zeros((16,), jnp.float32)
    for r in range(2):
        for i in range(HB // 16):
            contrib_v[r, pl.ds(i * 16, 16)] = zf16
    for r in range(8):
        for i in range(AW // 16):
            lcnt_v[r, pl.ds(i * 16, 16)] = zf16
            ltsum_v[r, pl.ds(i * 16, 16)] = zf16

    @pl.when(tid == 0)
    def _init_shared():
        for r in range(8):
            pltpu.sync_copy(zeros_hbm.at[pl.ds(0, 1), pl.ds(0, HB)],
                            acc_sh.at[pl.ds(r, 1)])

    pltpu.sync_copy(rowsel_hbm.at[pl.ds(tid * 8, 2)], rowsel_v)
    for k in range(K):
        pltpu.sync_copy(tabs_hbm.at[pl.ds(s0 + k, 1)], tabs_v.at[pl.ds(k, 1)])
        pltpu.sync_copy(addrt_hbm.at[pl.ds(s0 + k, 1), pl.ds(b0, HB)],
                        addrb_v.at[pl.ds(k, 1)])
    plsc.subcore_barrier()

    iota16 = lax.iota(jnp.int32, 16)
    UNR = 8
    for k in range(K):
        sg = s0 + k
        j = lax.rem(sg, NB)
        c = lax.div(sg, NB)
        wf = lax.shift_left(jnp.int32(1), j).astype(jnp.float32)
        kvec = jnp.full((16,), k, jnp.int32)
        cvec = jnp.broadcast_to(c - c_lo, (16,))

        def body(i, _):
            for u in range(UNR):
                off = i * (16 * UNR) + u * 16
                idx = addrb_v[k, pl.ds(off, 16)]
                v = plsc.load_gather(tabs_v, [kvec, idx])
                contrib = jnp.where(v > 0.5, wf, 0.0)
                plsc.addupdate_scatter(contrib_v, [cvec, off + iota16], contrib)
            return 0

        lax.fori_loop(0, HB // (16 * UNR), body, 0)

    pltpu.sync_copy(contrib_v, acc_sh.at[rowsel_v], add=True)
    plsc.subcore_barrier()

    aw0 = tid * AW
    ones16 = jnp.ones((16,), jnp.float32)
    for c in range(C):
        pltpu.sync_copy(acc_sh.at[pl.ds(c, 1)], ao_v.at[pl.ds(c, 1)])
        pltpu.sync_copy(tgt_hbm.at[pl.ds(c, 1), pl.ds(b0, HB)], tg_v.at[pl.ds(c, 1)])
    for c in range(C):
        ccol = jnp.full((16,), c, jnp.int32)

        def body2(i, _):
            for u in range(UNR):
                off = i * (16 * UNR) + u * 16
                rel = ao_v[c, pl.ds(off, 16)].astype(jnp.int32) - aw0
                m = jnp.logical_and(rel >= 0, rel < AW)
                relc = jnp.clip(rel, 0, AW - 1)
                plsc.addupdate_scatter(lcnt_v, [ccol, relc], ones16, mask=m)
                plsc.addupdate_scatter(ltsum_v, [ccol, relc],
                                       tg_v[c, pl.ds(off, 16)], mask=m)
            return 0

        lax.fori_loop(0, HB // (16 * UNR), body2, 0)

    for r in range(8):
        pltpu.sync_copy(lcnt_v.at[pl.ds(r, 1)],
                        cnt_sh.at[pl.ds(r, 1), pl.ds(aw0, AW)])
        pltpu.sync_copy(ltsum_v.at[pl.ds(r, 1)],
                        tsum_sh.at[pl.ds(r, 1), pl.ds(aw0, AW)])
    plsc.subcore_barrier()

    @pl.when(tid == 0)
    def _writeout():
        pltpu.sync_copy(cnt_sh, out_hbm.at[core * 2])
        pltpu.sync_copy(tsum_sh, out_hbm.at[core * 2 + 1])


def _combine_tc_kernel(om_ref, part_ref, out_ref):
    cnt = part_ref[0, :C, :] + part_ref[2, :C, :]
    ts = part_ref[1, :C, :] + part_ref[3, :C, :]
    om = om_ref[...]
    out_ref[...] = om * (1.0 - cnt) + ts


def kernel(target_bits, state_mem, out_mem, char_bits, prev_state, conn_state, conn_out):
    f32 = jnp.float32

    cof = conn_out.reshape(-1)
    cof128 = jnp.pad(cof, (0, S - cof.shape[0]))
    cof_row = cof128[None, :]
    cst = jnp.pad(conn_state.T.astype(f32), ((0, 4), (0, 0)))
    char_pad = jnp.pad(char_bits, ((0, 0), (0, 2))).astype(f32)
    tgt_t = jnp.pad(target_bits.T, ((0, 2), (0, 0)))
    rows8 = jnp.arange(8, dtype=jnp.int32)

    addrt, tabs80 = pl.pallas_call(
        _addr_tc_kernel,
        grid=(B // BB,),
        in_specs=[
            pl.BlockSpec((BB, N), lambda i: (i, 0)),
            pl.BlockSpec((BB, 8), lambda i: (i, 0)),
            pl.BlockSpec((16, N), lambda i: (0, 0)),
            pl.BlockSpec((1, S), lambda i: (0, 0)),
            pl.BlockSpec(memory_space=pltpu.MemorySpace.SMEM),
            pl.BlockSpec(memory_space=pltpu.MemorySpace.HBM),
        ],
        out_specs=[
            pl.BlockSpec((S, BB), lambda i: (0, i)),
            pl.BlockSpec((RPS, N), lambda i: (i, 0)),
        ],
        out_shape=[
            jax.ShapeDtypeStruct((S, B), jnp.int32),
            jax.ShapeDtypeStruct((S, N), f32),
        ],
        scratch_shapes=[
            pltpu.VMEM((N, S), jnp.bfloat16),
            pltpu.VMEM((N, S), jnp.bfloat16),
            pltpu.VMEM((8, S), f32),
            pltpu.SemaphoreType.DMA,
        ],
    )(prev_state, char_pad, cst, cof_row, cof128, state_mem)

    mesh = plsc.VectorSubcoreMesh(core_axis_name="c", subcore_axis_name="s")
    sc_call = pl.kernel(
        _sc_kernel,
        out_type=jax.ShapeDtypeStruct((4, 8, N), f32),
        mesh=mesh,
        compiler_params=pltpu.CompilerParams(use_tc_tiling_on_sc=False,
                                             needs_layout_passes=False),
        scratch_types=[
            pltpu.VMEM((K, N), f32),
            pltpu.VMEM((K, HB), jnp.int32),
            pltpu.VMEM((2, HB), f32),
            pltpu.VMEM((8, AW), f32),
            pltpu.VMEM((8, AW), f32),
            pltpu.VMEM((8, HB), f32),
            pltpu.VMEM((8, HB), f32),
            pltpu.VMEM((2,), jnp.int32),
            pltpu.VMEM_SHARED((8, HB), f32),
            pltpu.VMEM_SHARED((8, N), f32),
            pltpu.VMEM_SHARED((8, N), f32),
            pltpu.SemaphoreType.DMA,
        ],
    )
    tl = jnp.arange(NTILES, dtype=jnp.int32) * K // NB
    rowsel = jnp.stack([tl, jnp.minimum(tl + 1, 7)], axis=1)
    rowsel_pad = jnp.pad(rowsel, ((0, 0), (0, 6))).reshape(-1)
    zeros_half = jnp.zeros((1, HB), f32)
    partials = sc_call(addrt, tabs80, tgt_t, rowsel_pad, zeros_half)

    new_out = pl.pallas_call(
        _combine_tc_kernel,
        out_shape=jax.ShapeDtypeStruct((C, N), f32),
    )(out_mem, partials)
    return new_out


# speedup vs baseline: 2.0170x; 1.0165x over previous
import functools

import jax
import jax.numpy as jnp
from jax import lax
from jax.experimental import pallas as pl
from jax.experimental.pallas import tpu as pltpu
from jax.experimental.pallas import tpu_sc as plsc

B = 4096
N = 4096
C = 6
NB = 12
TIN = C + N
S = 128
SU = 80
BB = 1024
NTILES = 16
K = 5
HB = B // 2
TB = HB // NTILES


RPS = S // (B // BB)


def _addr_tc_kernel(prev_ref, char_ref, cst_ref, cof_ref, cof80_ref, smem_ref,
                    out_ref, tabs_ref, wlo_ref, whi_ref, wc_ref, sem):
    pid = pl.program_id(0)
    copies = [
        pltpu.make_async_copy(
            smem_ref.at[pl.ds(cof80_ref[pid * RPS + k], 1), :],
            tabs_ref.at[pl.ds(k, 1), :], sem)
        for k in range(RPS)
    ]
    for cp in copies:
        cp.start()

    @pl.when(pl.program_id(0) == 0)
    def _build_weights():
        niota = lax.broadcasted_iota(jnp.int32, (N, S), 0)
        m = (niota == cof_ref[...]).astype(jnp.float32)
        cselt = jnp.dot(cst_ref[...], m, preferred_element_type=jnp.float32,
                        precision=lax.Precision.HIGHEST).astype(jnp.int32)
        uio = lax.broadcasted_iota(jnp.int32, (8, S), 0)
        wlo = jnp.zeros((N, S), jnp.float32)
        whi = jnp.zeros((N, S), jnp.float32)
        wc = jnp.zeros((8, S), jnp.float32)
        for k in range(NB):
            row = cselt[k:k + 1, :]
            sm = (niota == row - C).astype(jnp.float32)
            if k < 8:
                wlo = wlo + float(2 ** k) * sm
            else:
                whi = whi + float(2 ** (k - 8)) * sm
            cm = jnp.logical_and(uio == row, uio < C)
            wc = wc + float(2 ** k) * cm.astype(jnp.float32)
        wlo_ref[...] = wlo.astype(jnp.bfloat16)
        whi_ref[...] = whi.astype(jnp.bfloat16)
        wc_ref[...] = wc

    pv = prev_ref[...].astype(jnp.bfloat16)
    dn = (((0,), (1,)), ((), ()))
    alo = lax.dot_general(wlo_ref[...], pv, dn, preferred_element_type=jnp.float32)
    ahi = lax.dot_general(whi_ref[...], pv, dn, preferred_element_type=jnp.float32)
    ac = lax.dot_general(wc_ref[...], char_ref[...], dn, preferred_element_type=jnp.float32,
                         precision=lax.Precision.HIGHEST)
    out_ref[...] = (alo + 256.0 * ahi + ac).astype(jnp.int32)
    for cp in copies:
        cp.wait()


AW = N // NTILES


def _sc_kernel(addrt_hbm, tabs_hbm, tgt_hbm, rowsel_hbm, zeros_hbm, out_hbm,
               tabs_v, addrb_v, contrib_v, lcnt_v, ltsum_v,
               ao_v, tg_v, rowsel_v, acc_sh, cnt_sh, tsum_sh, sem):
    core = lax.axis_index("c")
    tid = lax.axis_index("s")
    b0 = core * HB
    s0 = tid * K
    c_lo = lax.div(s0, NB)

    zf16 = jnp.zeros((16,), jnp.float32)
    for r in range(2):
        for i in range(HB // 16):
            contrib_v[r, pl.ds(i * 16, 16)] = zf16
    for r in range(8):
        for i in range(AW // 16):
            lcnt_v[r, pl.ds(i * 16, 16)] = zf16
            ltsum_v[r, pl.ds(i * 16, 16)] = zf16

    @pl.when(tid == 0)
    def _init_shared():
        for r in range(8):
            pltpu.sync_copy(zeros_hbm.at[pl.ds(0, 1), pl.ds(0, HB)],
                            acc_sh.at[pl.ds(r, 1)])

    pltpu.sync_copy(rowsel_hbm.at[pl.ds(tid * 8, 2)], rowsel_v)
    for k in range(K):
        pltpu.sync_copy(tabs_hbm.at[pl.ds(s0 + k, 1)], tabs_v.at[pl.ds(k, 1)])
        pltpu.sync_copy(addrt_hbm.at[pl.ds(s0 + k, 1), pl.ds(b0, HB)],
                        addrb_v.at[pl.ds(k, 1)])
    plsc.subcore_barrier()

    iota16 = lax.iota(jnp.int32, 16)
    UNR = 8
    for k in range(K):
        sg = s0 + k
        j = lax.rem(sg, NB)
        c = lax.div(sg, NB)
        wf = lax.shift_left(jnp.int32(1), j).astype(jnp.float32)
        kvec = jnp.full((16,), k, jnp.int32)
        cvec = jnp.broadcast_to(c - c_lo, (16,))

        def body(i, _):
            for u in range(UNR):
                off = i * (16 * UNR) + u * 16
                idx = addrb_v[k, pl.ds(off, 16)]
                v = plsc.load_gather(tabs_v, [kvec, idx])
                contrib = jnp.where(v > 0.5, wf, 0.0)
                plsc.addupdate_scatter(contrib_v, [cvec, off + iota16], contrib)
            return 0

        lax.fori_loop(0, HB // (16 * UNR), body, 0)

    pltpu.sync_copy(contrib_v, acc_sh.at[rowsel_v], add=True)
    plsc.subcore_barrier()

    aw0 = tid * AW
    ones16 = jnp.ones((16,), jnp.float32)
    for c in range(C):
        pltpu.sync_copy(acc_sh.at[pl.ds(c, 1)], ao_v.at[pl.ds(c, 1)])
        pltpu.sync_copy(tgt_hbm.at[pl.ds(c, 1), pl.ds(b0, HB)], tg_v.at[pl.ds(c, 1)])
    for c in range(C):
        ccol = jnp.full((16,), c, jnp.int32)

        def body2(i, _):
            for u in range(UNR):
                off = i * (16 * UNR) + u * 16
                rel = ao_v[c, pl.ds(off, 16)].astype(jnp.int32) - aw0
                m = jnp.logical_and(rel >= 0, rel < AW)
                relc = jnp.clip(rel, 0, AW - 1)
                plsc.addupdate_scatter(lcnt_v, [ccol, relc], ones16, mask=m)
                plsc.addupdate_scatter(ltsum_v, [ccol, relc],
                                       tg_v[c, pl.ds(off, 16)], mask=m)
            return 0

        lax.fori_loop(0, HB // (16 * UNR), body2, 0)

    for r in range(8):
        pltpu.sync_copy(lcnt_v.at[pl.ds(r, 1)],
                        cnt_sh.at[pl.ds(r, 1), pl.ds(aw0, AW)])
        pltpu.sync_copy(ltsum_v.at[pl.ds(r, 1)],
                        tsum_sh.at[pl.ds(r, 1), pl.ds(aw0, AW)])
    plsc.subcore_barrier()

    @pl.when(tid == 0)
    def _writeout():
        pltpu.sync_copy(cnt_sh, out_hbm.at[core * 2])
        pltpu.sync_copy(tsum_sh, out_hbm.at[core * 2 + 1])


def _combine_tc_kernel(om_ref, part_ref, out_ref):
    cnt = part_ref[0, :C, :] + part_ref[2, :C, :]
    ts = part_ref[1, :C, :] + part_ref[3, :C, :]
    om = om_ref[...]
    out_ref[...] = om * (1.0 - cnt) + ts


def kernel(target_bits, state_mem, out_mem, char_bits, prev_state, conn_state, conn_out):
    f32 = jnp.float32

    cof = conn_out.reshape(-1)
    cof128 = jnp.pad(cof, (0, S - cof.shape[0]))
    cof_row = cof128[None, :]
    cst = jnp.pad(conn_state.T.astype(f32), ((0, 4), (0, 0)))
    char_pad = jnp.pad(char_bits, ((0, 0), (0, 2))).astype(f32)
    tgt_t = jnp.pad(target_bits.T, ((0, 2), (0, 0)))
    rows8 = jnp.arange(8, dtype=jnp.int32)

    addrt, tabs80 = pl.pallas_call(
        _addr_tc_kernel,
        grid=(B // BB,),
        in_specs=[
            pl.BlockSpec((BB, N), lambda i: (i, 0)),
            pl.BlockSpec((BB, 8), lambda i: (i, 0)),
            pl.BlockSpec((16, N), lambda i: (0, 0)),
            pl.BlockSpec((1, S), lambda i: (0, 0)),
            pl.BlockSpec(memory_space=pltpu.MemorySpace.SMEM),
            pl.BlockSpec(memory_space=pltpu.MemorySpace.HBM),
        ],
        out_specs=[
            pl.BlockSpec((S, BB), lambda i: (0, i)),
            pl.BlockSpec((RPS, N), lambda i: (i, 0)),
        ],
        out_shape=[
            jax.ShapeDtypeStruct((S, B), jnp.int32),
            jax.ShapeDtypeStruct((S, N), f32),
        ],
        scratch_shapes=[
            pltpu.VMEM((N, S), jnp.bfloat16),
            pltpu.VMEM((N, S), jnp.bfloat16),
            pltpu.VMEM((8, S), f32),
            pltpu.SemaphoreType.DMA,
        ],
    )(prev_state, char_pad, cst, cof_row, cof128, state_mem)

    mesh = plsc.VectorSubcoreMesh(core_axis_name="c", subcore_axis_name="s")
    sc_call = pl.kernel(
        _sc_kernel,
        out_type=jax.ShapeDtypeStruct((4, 8, N), f32),
        mesh=mesh,
        compiler_params=pltpu.CompilerParams(use_tc_tiling_on_sc=False,
                                             needs_layout_passes=False),
        scratch_types=[
            pltpu.VMEM((K, N), f32),
            pltpu.VMEM((K, HB), jnp.int32),
            pltpu.VMEM((2, HB), f32),
            pltpu.VMEM((8, AW), f32),
            pltpu.VMEM((8, AW), f32),
            pltpu.VMEM((8, HB), f32),
            pltpu.VMEM((8, HB), f32),
            pltpu.VMEM((2,), jnp.int32),
            pltpu.VMEM_SHARED((8, HB), f32),
            pltpu.VMEM_SHARED((8, N), f32),
            pltpu.VMEM_SHARED((8, N), f32),
            pltpu.SemaphoreType.DMA,
        ],
    )
    tl = jnp.arange(NTILES, dtype=jnp.int32) * K // NB
    rowsel = jnp.stack([tl, jnp.minimum(tl + 1, 7)], axis=1)
    rowsel_pad = jnp.pad(rowsel, ((0, 0), (0, 6))).reshape(-1)
    zeros_half = jnp.zeros((1, HB), f32)
    partials = sc_call(addrt, tabs80, tgt_t, rowsel_pad, zeros_half)

    new_out = pl.pallas_call(
        _combine_tc_kernel,
        out_shape=jax.ShapeDtypeStruct((C, N), f32),
    )(out_mem, partials)
    return new_out
